# baseline (speedup 1.0000x reference)
import jax
import jax.numpy as jnp
from jax.experimental import pallas as pl
from jax.experimental.pallas import tpu as pltpu


_SEQ_BLOCK = 2048


def _add_kernel(x_ref, pos_ref, out_ref):
    out_ref[...] = x_ref[...] + pos_ref[...]


def kernel(x, pos_table):
    B, L, D = x.shape
    pe = pos_table[:L]
    grid = (D // 256, B)
    return pl.pallas_call(
        _add_kernel,
        grid=grid,
        in_specs=[
            pl.BlockSpec((1, L, 256), lambda d, b: (b, 0, d)),
            pl.BlockSpec((L, 256), lambda d, b: (0, d)),
        ],
        out_specs=pl.BlockSpec((1, L, 256), lambda d, b: (b, 0, d)),
        out_shape=jax.ShapeDtypeStruct((B, L, D), x.dtype),
    )(x, pe)


# speedup vs baseline: 1.0601x; 1.0601x over previous
import jax
import jax.numpy as jnp
from jax.experimental import pallas as pl
from jax.experimental.pallas import tpu as pltpu


_SEQ_BLOCK = 2048


def _add_kernel(x_ref, pos_ref, out_ref):
    out_ref[...] = x_ref[...] + pos_ref[...]


def kernel(x, pos_table):
    B, L, D = x.shape
    pe = pos_table[:L]
    grid = (D // 512, B)
    return pl.pallas_call(
        _add_kernel,
        grid=grid,
        in_specs=[
            pl.BlockSpec((1, L, 512), lambda d, b: (b, 0, d)),
            pl.BlockSpec((L, 512), lambda d, b: (0, d)),
        ],
        out_specs=pl.BlockSpec((1, L, 512), lambda d, b: (b, 0, d)),
        out_shape=jax.ShapeDtypeStruct((B, L, D), x.dtype),
    )(x, pe)


# speedup vs baseline: 1.0620x; 1.0018x over previous
import jax
import jax.numpy as jnp
from jax.experimental import pallas as pl


_D_BLOCK = 512


def _add_kernel(x_ref, pos_ref, out_ref):
    out_ref[...] = x_ref[...] + pos_ref[...]


def kernel(x, pos_table):
    B, L, D = x.shape
    pe = pos_table[:L]
    return pl.pallas_call(
        _add_kernel,
        grid=(D // _D_BLOCK, B),
        in_specs=[
            pl.BlockSpec((1, L, _D_BLOCK), lambda d, b: (b, 0, d)),
            pl.BlockSpec((L, _D_BLOCK), lambda d, b: (0, d)),
        ],
        out_specs=pl.BlockSpec((1, L, _D_BLOCK), lambda d, b: (b, 0, d)),
        out_shape=jax.ShapeDtypeStruct((B, L, D), x.dtype),
    )(x, pe)
